# baseline (speedup 1.0000x reference)
import functools

import jax
import jax.numpy as jnp
from jax import lax
from jax.experimental import pallas as pl
from jax.experimental.pallas import tpu as pltpu
from jax.experimental.pallas import tpu_sc as plsc

VOCAB = 1000000
DIM = 64
BATCH = 16384

_INFO = plsc.get_sparse_core_info()
_NC = _INFO.num_cores
_NS = _INFO.num_subcores
_NW = _NC * _NS
_BPW = BATCH // _NW
_CHUNK = 128
_NCHUNK = _BPW // _CHUNK
_L = 16
_UNROLL = 4

_mesh = plsc.VectorSubcoreMesh(core_axis_name="c", subcore_axis_name="s")


@functools.partial(
    pl.kernel,
    mesh=_mesh,
    compiler_params=pltpu.CompilerParams(
        needs_layout_passes=False, use_tc_tiling_on_sc=False
    ),
    out_type=jax.ShapeDtypeStruct((BATCH,), jnp.float32),
    scratch_types=[
        pltpu.VMEM((_NCHUNK, _CHUNK), jnp.int32),
        pltpu.VMEM((_NCHUNK, _CHUNK), jnp.int32),
        pltpu.VMEM((_BPW, DIM), jnp.float32),
        pltpu.VMEM((_BPW, DIM), jnp.float32),
        pltpu.VMEM((_BPW,), jnp.float32),
        pltpu.VMEM((_BPW,), jnp.float32),
        pltpu.VMEM((_BPW,), jnp.float32),
        pltpu.SemaphoreType.DMA,
    ],
)
def _glove_sc(ii_hbm, jj_hbm, wi_hbm, wj_hbm, bi_hbm, bj_hbm, out_hbm,
              ii_v, jj_v, wir_v, wjr_v, bir_v, bjr_v, out_v, sem):
    wid = lax.axis_index("s") * _NC + lax.axis_index("c")
    base = wid * _BPW

    pltpu.sync_copy(ii_hbm.at[pl.ds(wid * _NCHUNK, _NCHUNK)], ii_v)
    pltpu.sync_copy(jj_hbm.at[pl.ds(wid * _NCHUNK, _NCHUNK)], jj_v)

    copies = []
    for k in range(_NCHUNK):
        rows = pl.ds(k * _CHUNK, _CHUNK)
        copies.append(pltpu.async_copy(wi_hbm.at[ii_v.at[k]], wir_v.at[rows], sem))
        copies.append(pltpu.async_copy(wj_hbm.at[jj_v.at[k]], wjr_v.at[rows], sem))
        copies.append(pltpu.async_copy(bi_hbm.at[ii_v.at[k]], bir_v.at[rows], sem))
        copies.append(pltpu.async_copy(bj_hbm.at[jj_v.at[k]], bjr_v.at[rows], sem))
    for c in copies:
        c.wait()

    iota = lax.iota(jnp.int32, _L)
    zeros_i = jnp.zeros((_L,), jnp.int32)

    def block(g, carry):
        rbase = g * _L
        row = iota + rbase
        acc = bir_v[pl.ds(rbase, _L)] + bjr_v[pl.ds(rbase, _L)]
        col = zeros_i
        for _ in range(DIM):
            a = plsc.load_gather(wir_v, [row, col])
            b = plsc.load_gather(wjr_v, [row, col])
            acc = acc + a * b
            col = col + 1
        out_v[pl.ds(rbase, _L)] = acc
        return carry

    lax.fori_loop(0, _BPW // _L, block, 0)

    pltpu.sync_copy(out_v, out_hbm.at[pl.ds(base, _BPW)])


def kernel(i_indices, j_indices, wi, wj, bi, bj):
    ii = i_indices.astype(jnp.int32).reshape(_NW * _NCHUNK, _CHUNK)
    jj = j_indices.astype(jnp.int32).reshape(_NW * _NCHUNK, _CHUNK)
    return _glove_sc(ii, jj, wi, wj, bi.reshape(VOCAB), bj.reshape(VOCAB))
